# baseline (speedup 1.0000x reference)
import functools

import numpy as np
import jax
import jax.numpy as jnp
from jax import lax
from jax.experimental import pallas as pl
from jax.experimental.pallas import tpu as pltpu
from jax.experimental.pallas import tpu_sc as plsc

_C, _T, _H, _W = 3, 32, 224, 224
_TS = _T // 4
_D = _H * _W
_IDX = tuple(int(v) for v in np.linspace(0.0, _T - 1, _TS))

_NC, _NS = 2, 16
_NW = _NC * _NS
_CHUNKS = 4
_CS = _D // _CHUNKS
_ROWS = _C * _TS
_ITEMS = _ROWS * _CHUNKS

_mesh = plsc.VectorSubcoreMesh(core_axis_name="c", subcore_axis_name="s")


@functools.partial(
    pl.kernel,
    mesh=_mesh,
    out_type=jax.ShapeDtypeStruct((_ROWS, _D), jnp.float32),
    scratch_types=[pltpu.SemaphoreType.DMA],
)
def _slow_gather(in_hbm, out_hbm, sem):
    wid = lax.axis_index("s") * _NC + lax.axis_index("c")
    for w in range(_NW):
        items = [i for i in range(_ITEMS) if i % _NW == w]

        @pl.when(wid == w)
        def _copies(items=items):
            descs = []
            for i in items:
                row, chunk = divmod(i, _CHUNKS)
                c, s = divmod(row, _TS)
                src_row = c * _T + _IDX[s]
                off = chunk * _CS
                descs.append(pltpu.make_async_copy(
                    in_hbm.at[src_row, pl.ds(off, _CS)],
                    out_hbm.at[row, pl.ds(off, _CS)],
                    sem))
            for d in descs:
                d.start()
            for d in descs:
                d.wait()


def kernel(frames):
    flat = frames.reshape(_C * _T, _D)
    slow = _slow_gather(flat).reshape(_C, _TS, _H, _W)
    return (slow, frames)


# speedup vs baseline: 3.1003x; 3.1003x over previous
import functools

import numpy as np
import jax
import jax.numpy as jnp
from jax import lax
from jax.experimental import pallas as pl
from jax.experimental.pallas import tpu as pltpu
from jax.experimental.pallas import tpu_sc as plsc

_C, _T, _H, _W = 3, 32, 224, 224
_TS = _T // 4
_D = _H * _W
_IDX = tuple(int(v) for v in np.linspace(0.0, _T - 1, _TS))

_NC, _NS = 2, 16
_NW = _NC * _NS
_CHUNKS = 4
_CS = _D // _CHUNKS
_ROWS = _C * _TS
_ITEMS = _ROWS * _CHUNKS

_mesh = plsc.VectorSubcoreMesh(core_axis_name="c", subcore_axis_name="s")


_PER_W = _ITEMS // _NW


@functools.partial(
    pl.kernel,
    mesh=_mesh,
    out_type=jax.ShapeDtypeStruct((_ROWS, _D), jnp.float32),
    scratch_types=[
        pltpu.VMEM((_PER_W * _CS,), jnp.float32),
        pltpu.SemaphoreType.DMA,
        pltpu.SemaphoreType.DMA,
    ],
)
def _slow_gather(in_hbm, out_hbm, buf, in_sem, out_sem):
    wid = lax.axis_index("s") * _NC + lax.axis_index("c")
    for w in range(_NW):
        items = [i for i in range(_ITEMS) if i % _NW == w]

        @pl.when(wid == w)
        def _copies(items=items):
            ins, outs = [], []
            for k, i in enumerate(items):
                row, chunk = divmod(i, _CHUNKS)
                c, s = divmod(row, _TS)
                src_row = c * _T + _IDX[s]
                off = chunk * _CS
                ins.append(pltpu.make_async_copy(
                    in_hbm.at[src_row, pl.ds(off, _CS)],
                    buf.at[pl.ds(k * _CS, _CS)], in_sem))
                outs.append(pltpu.make_async_copy(
                    buf.at[pl.ds(k * _CS, _CS)],
                    out_hbm.at[row, pl.ds(off, _CS)], out_sem))
            for d in ins:
                d.start()
            for din, dout in zip(ins, outs):
                din.wait()
                dout.start()
            for d in outs:
                d.wait()


def kernel(frames):
    flat = frames.reshape(_C * _T, _D)
    slow = _slow_gather(flat).reshape(_C, _TS, _H, _W)
    return (slow, frames)
